# baseline (speedup 1.0000x reference)
import functools

import jax
import jax.numpy as jnp
import numpy as np
from jax import lax
from jax.experimental import pallas as pl
from jax.experimental.pallas import tpu as pltpu
from jax.experimental.pallas import tpu_sc as plsc

NU = 50000
NI = 50000
D = 32
N = NU + NI
E = 1600000
B = 4096
NLAYERS = 3

NC = 2
NS = 16
LANES = 16
HALF = D // NC
CH = 128
HB = 6
NB = 2 * HB
NCHT = -(-E // (NS * CH * NB)) * NB
EPT = NCHT * CH
EPAD = EPT * NS
NP = 100096
RPT = NP // NS
ZR = 3128
BPT = B // NS


def _sc_body(emb, egdata, esdata, bidx, zeros, usum, psum, nsum,
             l1, l2, l3, acc, *scratch):
    c = lax.axis_index("c")
    s = lax.axis_index("s")

    gbuf = scratch[0:NB]
    sbuf = scratch[NB:2 * NB]
    rows = scratch[2 * NB:3 * NB]
    semL = scratch[3 * NB:3 * NB + 2]
    semG = scratch[3 * NB + 2:3 * NB + 4]
    semS = scratch[3 * NB + 4:3 * NB + 6]
    semg = scratch[3 * NB + 6]
    srcs = (emb.at[c], l1.at[c], l2.at[c], l3.at[c])
    dsts = (l1, l2, l3)

    def issue_lin(h, base):
        for j in range(HB):
            slot = h * HB + j
            gc = s * NCHT + base + j
            pltpu.async_copy(egdata.at[gc], gbuf[slot], semL[h])
            pltpu.async_copy(esdata.at[gc], sbuf[slot], semL[h])

    def wait_lin(h):
        for j in range(HB):
            slot = h * HB + j
            pltpu.make_async_copy(egdata.at[0], gbuf[slot], semL[h]).wait()
            pltpu.make_async_copy(esdata.at[0], sbuf[slot], semL[h]).wait()

    def issue_gathers(h, src):
        for j in range(HB):
            slot = h * HB + j
            pltpu.async_copy(src.at[gbuf[slot].at[0]], rows[slot], semG[h])

    def wait_gathers(h, src):
        for j in range(HB):
            slot = h * HB + j
            pltpu.make_async_copy(src.at[gbuf[slot].at[0]], rows[slot],
                                  semG[h]).wait()

    def issue_scatters(h):
        for j in range(HB):
            slot = h * HB + j
            pltpu.async_copy(rows[slot], acc.at[sbuf[slot]], semS[h],
                             add=True)

    def wait_scatters(h):
        for j in range(HB):
            slot = h * HB + j
            pltpu.make_async_copy(rows[slot], acc.at[sbuf[slot]],
                                  semS[h]).wait()

    def scale_half(h):
        for j in range(HB):
            slot = h * HB + j
            gb = gbuf[slot]
            rr = rows[slot]

            @pl.loop(0, CH // LANES)
            def _scale(g):
                vvec = lax.bitcast_convert_type(gb[1, pl.ds(g * LANES, LANES)],
                                                jnp.float32)
                for e0 in range(LANES):
                    e = g * LANES + e0
                    rr[e] = rr[e] * vvec[e0]

    for l in range(NLAYERS):
        src = srcs[l]
        dst = dsts[l]
        for k in range(RPT // ZR):
            pltpu.sync_copy(zeros, acc.at[pl.ds(s * RPT + k * ZR, ZR)])
        plsc.subcore_barrier()

        issue_lin(0, 0)
        issue_lin(1, HB)
        wait_lin(0)
        issue_gathers(0, src)

        @pl.loop(0, NCHT, step=NB)
        def _edge_group(g0):
            wait_gathers(0, src)
            scale_half(0)
            issue_scatters(0)
            wait_lin(1)
            issue_gathers(1, src)
            wait_scatters(0)

            @pl.when(g0 + NB < NCHT)
            def _():
                issue_lin(0, g0 + NB)

            wait_gathers(1, src)
            scale_half(1)
            issue_scatters(1)

            @pl.when(g0 + NB < NCHT)
            def _():
                wait_lin(0)
                issue_gathers(0, src)

            wait_scatters(1)

            @pl.when(g0 + NB < NCHT)
            def _():
                issue_lin(1, g0 + NB + HB)

        plsc.subcore_barrier()
        for k in range(RPT // ZR):
            r0 = s * RPT + k * ZR
            pltpu.sync_copy(acc.at[pl.ds(r0, ZR)], dst.at[c, pl.ds(r0, ZR)])
        plsc.subcore_barrier()

    stages = rows[0:4]
    idxb = gbuf[0].at[0]
    outs = (usum, psum, nsum)
    for cc in range(BPT // CH):
        b0 = s * BPT + cc * CH
        for t in range(3):
            pltpu.sync_copy(bidx.at[pl.ds(t * B + b0, CH)], idxb)
            cps = [pltpu.async_copy(srcs[l].at[idxb], stages[l], semg)
                   for l in range(4)]
            for cp in cps:
                cp.wait()

            @pl.loop(0, CH, unroll=8)
            def _sum_rows(e):
                stages[0][e] = ((stages[0][e] + stages[1][e]) +
                                (stages[2][e] + stages[3][e]))

            pltpu.sync_copy(stages[0], outs[t].at[c, pl.ds(b0, CH)])


_SLOT_SCRATCH = (
    [pltpu.VMEM((2, CH), jnp.int32) for _ in range(NB)]
    + [pltpu.VMEM((CH,), jnp.int32) for _ in range(NB)]
    + [pltpu.VMEM((CH, HALF), jnp.float32) for _ in range(NB)]
    + [pltpu.SemaphoreType.DMA for _ in range(6)]
)


@functools.partial(
    pl.kernel,
    out_type=[
        jax.ShapeDtypeStruct((NC, B, HALF), jnp.float32),
        jax.ShapeDtypeStruct((NC, B, HALF), jnp.float32),
        jax.ShapeDtypeStruct((NC, B, HALF), jnp.float32),
        jax.ShapeDtypeStruct((NC, NP, HALF), jnp.float32),
        jax.ShapeDtypeStruct((NC, NP, HALF), jnp.float32),
        jax.ShapeDtypeStruct((NC, NP, HALF), jnp.float32),
    ],
    mesh=plsc.VectorSubcoreMesh(core_axis_name="c", subcore_axis_name="s"),
    compiler_params=pltpu.CompilerParams(use_tc_tiling_on_sc=False),
    scratch_types=[pltpu.VMEM_SHARED((NP, HALF), jnp.float32)]
    + _SLOT_SCRATCH
    + [pltpu.SemaphoreType.DMA],
)
def _sc_propagate(emb, egdata, esdata, bidx, zeros, usum, psum, nsum,
                  l1, l2, l3, acc, *scratch):
    _sc_body(emb, egdata, esdata, bidx, zeros, usum, psum, nsum, l1, l2, l3,
             acc, *scratch)


def _loss_body(u_ref, p_ref, n_ref, o_ref):
    u = u_ref[...]
    p = p_ref[...]
    n = n_ref[...]
    t = u * (p - n)
    rowx = jnp.sum(t, axis=1)
    x = (rowx[:B] + rowx[B:]) * (1.0 / 16.0)
    z = jnp.minimum(x, 0.0) - jnp.log(1.0 + jnp.exp(-jnp.abs(x)))
    o_ref[...] = jnp.reshape(-jnp.sum(z) * (1.0 / B), (1, 1))


_loss_call = pl.pallas_call(
    _loss_body,
    out_shape=jax.ShapeDtypeStruct((1, 1), jnp.float32),
)


def kernel(user_table, item_table, adj_val, adj_row, adj_col,
           user_idx, pos_item_idx, neg_item_idx):
    all_emb = jnp.concatenate([user_table, item_table], axis=0)
    halves = jnp.stack([all_emb[:, :HALF], all_emb[:, HALF:]])
    pad = EPAD - E
    spread = jnp.arange(pad, dtype=jnp.int32) % N
    colp = jnp.concatenate([adj_col, spread])
    rowp = jnp.concatenate([adj_row, spread])
    valp = jnp.concatenate([adj_val, jnp.zeros((pad,), jnp.float32)])
    egdata = jnp.stack([jnp.reshape(colp, (-1, CH)),
                        jnp.reshape(valp, (-1, CH)).view(jnp.int32)], axis=1)
    esdata = jnp.reshape(rowp, (-1, CH))
    bidx = jnp.concatenate([user_idx, pos_item_idx + NU, neg_item_idx + NU])
    zeros = jnp.zeros((ZR, HALF), jnp.float32)
    usum, psum, nsum, _, _, _ = _sc_propagate(halves, egdata, esdata, bidx,
                                              zeros)
    loss = _loss_call(jnp.reshape(usum, (NC * B, HALF)),
                      jnp.reshape(psum, (NC * B, HALF)),
                      jnp.reshape(nsum, (NC * B, HALF)))
    return loss[0, 0]


# speedup vs baseline: 1.2245x; 1.2245x over previous
import functools

import jax
import jax.numpy as jnp
import numpy as np
from jax import lax
from jax.experimental import pallas as pl
from jax.experimental.pallas import tpu as pltpu
from jax.experimental.pallas import tpu_sc as plsc

NU = 50000
NI = 50000
D = 32
N = NU + NI
E = 1600000
B = 4096
NLAYERS = 3

NC = 2
NS = 16
LANES = 16
HALF = D // NC
CH = 128
NB = 8
NCHT = -(-E // (NS * CH * NB)) * NB
EPT = NCHT * CH
EPAD = EPT * NS
NP = 100096
RPT = NP // NS
ZR = 3128
BPT = B // NS


def _splat_i(v):
    return lax.full((LANES,), v, jnp.int32)


def _sc_body(emb, egdata, esdata, bidx, zeros, usum, psum, nsum,
             l1, l2, l3, acc, *scratch):
    c = lax.axis_index("c")
    s = lax.axis_index("s")

    gbuf = scratch[0:NB]
    sbuf = scratch[NB:2 * NB]
    rows = scratch[2 * NB:3 * NB]
    semL = scratch[3 * NB:4 * NB]
    semG = scratch[4 * NB:5 * NB]
    semS = scratch[5 * NB:6 * NB]
    semg = scratch[6 * NB]
    srcs = (emb.at[c], l1.at[c], l2.at[c], l3.at[c])
    dsts = (l1, l2, l3)

    def issue_lin(b, cidx):
        gc = s * NCHT + cidx
        pltpu.async_copy(egdata.at[gc], gbuf[b], semL[b])
        pltpu.async_copy(esdata.at[gc], sbuf[b], semL[b])

    def wait_lin(b):
        pltpu.make_async_copy(egdata.at[0], gbuf[b], semL[b]).wait()
        pltpu.make_async_copy(esdata.at[0], sbuf[b], semL[b]).wait()

    def scale_slot(b):
        gb = gbuf[b]
        rr = rows[b]

        @pl.loop(0, CH // LANES)
        def _scale(g):
            vvec = lax.bitcast_convert_type(gb[1, pl.ds(g * LANES, LANES)],
                                            jnp.float32)
            for e0 in range(LANES):
                e = g * LANES + e0
                bc = lax.gather(
                    vvec, _splat_i(e0)[:, None],
                    lax.GatherDimensionNumbers(
                        offset_dims=(), collapsed_slice_dims=(0,),
                        start_index_map=(0,)),
                    slice_sizes=(1,),
                    mode=lax.GatherScatterMode.PROMISE_IN_BOUNDS)
                rr[e] = rr[e] * bc

    for l in range(NLAYERS):
        src = srcs[l]
        dst = dsts[l]
        for k in range(RPT // ZR):
            pltpu.sync_copy(zeros, acc.at[pl.ds(s * RPT + k * ZR, ZR)])
        plsc.subcore_barrier()

        for b in range(NB):
            issue_lin(b, b)

        @pl.loop(0, NCHT, step=NB)
        def _edge_group(g0):
            for b in range(NB):
                wait_lin(b)
                pltpu.async_copy(src.at[gbuf[b].at[0]], rows[b], semG[b])
            for b in range(NB):
                pltpu.make_async_copy(src.at[gbuf[b].at[0]], rows[b],
                                      semG[b]).wait()
                scale_slot(b)
                pltpu.async_copy(rows[b], acc.at[sbuf[b]], semS[b], add=True)
            for b in range(NB):
                pltpu.make_async_copy(rows[b], acc.at[sbuf[b]], semS[b]).wait()

                @pl.when(g0 + NB + b < NCHT)
                def _():
                    issue_lin(b, g0 + NB + b)

        plsc.subcore_barrier()
        for k in range(RPT // ZR):
            r0 = s * RPT + k * ZR
            pltpu.sync_copy(acc.at[pl.ds(r0, ZR)], dst.at[c, pl.ds(r0, ZR)])
        plsc.subcore_barrier()

    stages = rows[0:4]
    idxb = gbuf[0].at[0]
    outs = (usum, psum, nsum)
    for cc in range(BPT // CH):
        b0 = s * BPT + cc * CH
        for t in range(3):
            pltpu.sync_copy(bidx.at[pl.ds(t * B + b0, CH)], idxb)
            cps = [pltpu.async_copy(srcs[l].at[idxb], stages[l], semg)
                   for l in range(4)]
            for cp in cps:
                cp.wait()

            @pl.loop(0, CH, unroll=8)
            def _sum_rows(e):
                stages[0][e] = ((stages[0][e] + stages[1][e]) +
                                (stages[2][e] + stages[3][e]))

            pltpu.sync_copy(stages[0], outs[t].at[c, pl.ds(b0, CH)])


_SLOT_SCRATCH = (
    [pltpu.VMEM((2, CH), jnp.int32) for _ in range(NB)]
    + [pltpu.VMEM((CH,), jnp.int32) for _ in range(NB)]
    + [pltpu.VMEM((CH, HALF), jnp.float32) for _ in range(NB)]
    + [pltpu.SemaphoreType.DMA for _ in range(3 * NB)]
)


@functools.partial(
    pl.kernel,
    out_type=[
        jax.ShapeDtypeStruct((NC, B, HALF), jnp.float32),
        jax.ShapeDtypeStruct((NC, B, HALF), jnp.float32),
        jax.ShapeDtypeStruct((NC, B, HALF), jnp.float32),
        jax.ShapeDtypeStruct((NC, NP, HALF), jnp.float32),
        jax.ShapeDtypeStruct((NC, NP, HALF), jnp.float32),
        jax.ShapeDtypeStruct((NC, NP, HALF), jnp.float32),
    ],
    mesh=plsc.VectorSubcoreMesh(core_axis_name="c", subcore_axis_name="s"),
    compiler_params=pltpu.CompilerParams(use_tc_tiling_on_sc=False),
    scratch_types=[pltpu.VMEM_SHARED((NP, HALF), jnp.float32)]
    + _SLOT_SCRATCH
    + [pltpu.SemaphoreType.DMA],
)
def _sc_propagate(emb, egdata, esdata, bidx, zeros, usum, psum, nsum,
                  l1, l2, l3, acc, *scratch):
    _sc_body(emb, egdata, esdata, bidx, zeros, usum, psum, nsum, l1, l2, l3,
             acc, *scratch)


def _loss_body(u_ref, p_ref, n_ref, o_ref):
    u = u_ref[...]
    p = p_ref[...]
    n = n_ref[...]
    t = u * (p - n)
    rowx = jnp.sum(t, axis=1)
    x = (rowx[:B] + rowx[B:]) * (1.0 / 16.0)
    z = jnp.minimum(x, 0.0) - jnp.log(1.0 + jnp.exp(-jnp.abs(x)))
    o_ref[...] = jnp.reshape(-jnp.sum(z) * (1.0 / B), (1, 1))


_loss_call = pl.pallas_call(
    _loss_body,
    out_shape=jax.ShapeDtypeStruct((1, 1), jnp.float32),
)


def kernel(user_table, item_table, adj_val, adj_row, adj_col,
           user_idx, pos_item_idx, neg_item_idx):
    all_emb = jnp.concatenate([user_table, item_table], axis=0)
    halves = jnp.stack([all_emb[:, :HALF], all_emb[:, HALF:]])
    pad = EPAD - E
    spread = jnp.arange(pad, dtype=jnp.int32) % N
    colp = jnp.concatenate([adj_col, spread])
    rowp = jnp.concatenate([adj_row, spread])
    valp = jnp.concatenate([adj_val, jnp.zeros((pad,), jnp.float32)])
    egdata = jnp.stack([jnp.reshape(colp, (-1, CH)),
                        jnp.reshape(valp, (-1, CH)).view(jnp.int32)], axis=1)
    esdata = jnp.reshape(rowp, (-1, CH))
    bidx = jnp.concatenate([user_idx, pos_item_idx + NU, neg_item_idx + NU])
    zeros = jnp.zeros((ZR, HALF), jnp.float32)
    usum, psum, nsum, _, _, _ = _sc_propagate(halves, egdata, esdata, bidx,
                                              zeros)
    loss = _loss_call(jnp.reshape(usum, (NC * B, HALF)),
                      jnp.reshape(psum, (NC * B, HALF)),
                      jnp.reshape(nsum, (NC * B, HALF)))
    return loss[0, 0]


# speedup vs baseline: 1.2266x; 1.0017x over previous
import functools

import jax
import jax.numpy as jnp
import numpy as np
from jax import lax
from jax.experimental import pallas as pl
from jax.experimental.pallas import tpu as pltpu
from jax.experimental.pallas import tpu_sc as plsc

NU = 50000
NI = 50000
D = 32
N = NU + NI
E = 1600000
B = 4096
NLAYERS = 3

NC = 2
NS = 16
LANES = 16
HALF = D // NC
CH = 128
NB = 8
NCHT = -(-E // (NS * CH * NB)) * NB
EPT = NCHT * CH
EPAD = EPT * NS
NP = 100096
RPT = NP // NS
ZR = 3128
BPT = B // NS


def _splat_i(v):
    return lax.full((LANES,), v, jnp.int32)


def _sc_body(emb, egdata, esdata, bidx, zeros, usum, psum, nsum,
             l1, l2, l3, acc, *scratch):
    c = lax.axis_index("c")
    s = lax.axis_index("s")

    gbuf = scratch[0:NB]
    sbuf = scratch[NB:2 * NB]
    rows = scratch[2 * NB:3 * NB]
    semL = scratch[3 * NB:4 * NB]
    semG = scratch[4 * NB:5 * NB]
    semS = scratch[5 * NB:6 * NB]
    semg = scratch[6 * NB]
    srcs = (emb.at[c], l1.at[c], l2.at[c], l3.at[c])
    dsts = (l1, l2, l3)

    def issue_lin(b, cidx):
        gc = s * NCHT + cidx
        pltpu.async_copy(egdata.at[gc], gbuf[b], semL[b])
        pltpu.async_copy(esdata.at[gc], sbuf[b], semL[b])

    def wait_lin(b):
        pltpu.make_async_copy(egdata.at[0], gbuf[b], semL[b]).wait()
        pltpu.make_async_copy(esdata.at[0], sbuf[b], semL[b]).wait()

    def scale_slot(b):
        gb = gbuf[b]
        rr = rows[b]

        @pl.loop(0, CH // LANES)
        def _scale(g):
            vvec = lax.bitcast_convert_type(gb[1, pl.ds(g * LANES, LANES)],
                                            jnp.float32)
            for e0 in range(LANES):
                e = g * LANES + e0
                bc = lax.gather(
                    vvec, _splat_i(e0)[:, None],
                    lax.GatherDimensionNumbers(
                        offset_dims=(), collapsed_slice_dims=(0,),
                        start_index_map=(0,)),
                    slice_sizes=(1,),
                    mode=lax.GatherScatterMode.PROMISE_IN_BOUNDS)
                rr[e] = rr[e] * bc

    for l in range(NLAYERS):
        src = srcs[l]
        dst = dsts[l]
        if l == 0:
            for k in range(RPT // ZR):
                pltpu.sync_copy(zeros, acc.at[pl.ds(s * RPT + k * ZR, ZR)])
            plsc.subcore_barrier()

        for b in range(NB):
            issue_lin(b, b)

        @pl.loop(0, NCHT, step=NB)
        def _edge_group(g0):
            for b in range(NB):
                wait_lin(b)
                pltpu.async_copy(src.at[gbuf[b].at[0]], rows[b], semG[b])
            for b in range(NB):
                pltpu.make_async_copy(src.at[gbuf[b].at[0]], rows[b],
                                      semG[b]).wait()
                scale_slot(b)
                pltpu.async_copy(rows[b], acc.at[sbuf[b]], semS[b], add=True)
            for b in range(NB):
                pltpu.make_async_copy(rows[b], acc.at[sbuf[b]], semS[b]).wait()

                @pl.when(g0 + NB + b < NCHT)
                def _():
                    issue_lin(b, g0 + NB + b)

        plsc.subcore_barrier()
        for k in range(RPT // ZR):
            r0 = s * RPT + k * ZR
            pltpu.sync_copy(acc.at[pl.ds(r0, ZR)], dst.at[c, pl.ds(r0, ZR)])
        if l < NLAYERS - 1:
            for k in range(RPT // ZR):
                pltpu.sync_copy(zeros, acc.at[pl.ds(s * RPT + k * ZR, ZR)])
        plsc.subcore_barrier()

    bsrcs = (emb.at[c], l1.at[c], l2.at[c], l3.at[c])
    stages = rows[0:4]
    idxb = gbuf[0].at[0]
    outs = (usum, psum, nsum)
    for cc in range(BPT // CH):
        b0 = s * BPT + cc * CH
        for t in range(3):
            pltpu.sync_copy(bidx.at[pl.ds(t * B + b0, CH)], idxb)
            cps = [pltpu.async_copy(bsrcs[l].at[idxb], stages[l], semg)
                   for l in range(4)]
            for cp in cps:
                cp.wait()

            @pl.loop(0, CH, unroll=8)
            def _sum_rows(e):
                stages[0][e] = ((stages[0][e] + stages[1][e]) +
                                (stages[2][e] + stages[3][e]))

            pltpu.sync_copy(stages[0], outs[t].at[c, pl.ds(b0, CH)])


_SLOT_SCRATCH = (
    [pltpu.VMEM((2, CH), jnp.int32) for _ in range(NB)]
    + [pltpu.VMEM((CH,), jnp.int32) for _ in range(NB)]
    + [pltpu.VMEM((CH, HALF), jnp.float32) for _ in range(NB)]
    + [pltpu.SemaphoreType.DMA for _ in range(3 * NB)]
)


@functools.partial(
    pl.kernel,
    out_type=[
        jax.ShapeDtypeStruct((NC, B, HALF), jnp.float32),
        jax.ShapeDtypeStruct((NC, B, HALF), jnp.float32),
        jax.ShapeDtypeStruct((NC, B, HALF), jnp.float32),
        jax.ShapeDtypeStruct((NC, NP, HALF), jnp.float32),
        jax.ShapeDtypeStruct((NC, NP, HALF), jnp.float32),
        jax.ShapeDtypeStruct((NC, NP, HALF), jnp.float32),
    ],
    mesh=plsc.VectorSubcoreMesh(core_axis_name="c", subcore_axis_name="s"),
    compiler_params=pltpu.CompilerParams(use_tc_tiling_on_sc=False),
    scratch_types=[pltpu.VMEM_SHARED((NP, HALF), jnp.float32)]
    + _SLOT_SCRATCH
    + [pltpu.SemaphoreType.DMA],
)
def _sc_propagate(emb, egdata, esdata, bidx, zeros, usum, psum, nsum,
                  l1, l2, l3, acc, *scratch):
    _sc_body(emb, egdata, esdata, bidx, zeros, usum, psum, nsum, l1, l2, l3,
             acc, *scratch)


def _loss_body(u_ref, p_ref, n_ref, o_ref):
    u = u_ref[...]
    p = p_ref[...]
    n = n_ref[...]
    t = u * (p - n)
    rowx = jnp.sum(t, axis=1)
    x = (rowx[:B] + rowx[B:]) * (1.0 / 16.0)
    z = jnp.minimum(x, 0.0) - jnp.log(1.0 + jnp.exp(-jnp.abs(x)))
    o_ref[...] = jnp.reshape(-jnp.sum(z) * (1.0 / B), (1, 1))


_loss_call = pl.pallas_call(
    _loss_body,
    out_shape=jax.ShapeDtypeStruct((1, 1), jnp.float32),
)


def kernel(user_table, item_table, adj_val, adj_row, adj_col,
           user_idx, pos_item_idx, neg_item_idx):
    all_emb = jnp.concatenate([user_table, item_table], axis=0)
    halves = jnp.stack([all_emb[:, :HALF], all_emb[:, HALF:]])
    pad = EPAD - E
    spread = jnp.arange(pad, dtype=jnp.int32) % N
    colp = jnp.concatenate([adj_col, spread])
    rowp = jnp.concatenate([adj_row, spread])
    valp = jnp.concatenate([adj_val, jnp.zeros((pad,), jnp.float32)])
    egdata = jnp.stack([jnp.reshape(colp, (-1, CH)),
                        jnp.reshape(valp, (-1, CH)).view(jnp.int32)], axis=1)
    esdata = jnp.reshape(rowp, (-1, CH))
    bidx = jnp.concatenate([user_idx, pos_item_idx + NU, neg_item_idx + NU])
    zeros = jnp.zeros((ZR, HALF), jnp.float32)
    usum, psum, nsum, _, _, _ = _sc_propagate(halves, egdata, esdata, bidx,
                                              zeros)
    loss = _loss_call(jnp.reshape(usum, (NC * B, HALF)),
                      jnp.reshape(psum, (NC * B, HALF)),
                      jnp.reshape(nsum, (NC * B, HALF)))
    return loss[0, 0]


# speedup vs baseline: 1.2272x; 1.0005x over previous
import functools

import jax
import jax.numpy as jnp
import numpy as np
from jax import lax
from jax.experimental import pallas as pl
from jax.experimental.pallas import tpu as pltpu
from jax.experimental.pallas import tpu_sc as plsc

NU = 50000
NI = 50000
D = 32
N = NU + NI
E = 1600000
B = 4096
NLAYERS = 3

NC = 2
NS = 16
LANES = 16
HALF = D // NC
CH = 128
NB = 8
NCHT = -(-E // (NS * CH * NB)) * NB
EPT = NCHT * CH
EPAD = EPT * NS
NP = 100096
RPT = NP // NS
ZR = 3128
BPT = B // NS


def _splat_i(v):
    return lax.full((LANES,), v, jnp.int32)


def _sc_body(emb, egdata, esdata, bidx, zeros, usum, psum, nsum,
             l1, l2, l3, acc, *scratch):
    c = lax.axis_index("c")
    s = lax.axis_index("s")

    gbuf = scratch[0:NB]
    sbuf = scratch[NB:2 * NB]
    rows = scratch[2 * NB:3 * NB]
    semL = scratch[3 * NB:4 * NB]
    semG = scratch[4 * NB:5 * NB]
    semS = scratch[5 * NB:6 * NB]
    semg = scratch[6 * NB]
    srcs = (emb.at[c], l1.at[c], l2.at[c], l3.at[c])
    dsts = (l1, l2, l3)

    def issue_lin(b, cidx):
        gc = s * NCHT + cidx
        pltpu.async_copy(egdata.at[gc], gbuf[b], semL[b])
        pltpu.async_copy(esdata.at[gc], sbuf[b], semL[b])

    def wait_lin(b):
        pltpu.make_async_copy(egdata.at[0], gbuf[b], semL[b]).wait()
        pltpu.make_async_copy(esdata.at[0], sbuf[b], semL[b]).wait()

    def scale_slot(b):
        gb = gbuf[b]
        rr = rows[b]

        @pl.loop(0, CH // LANES)
        def _scale(g):
            vvec = lax.bitcast_convert_type(gb[1, pl.ds(g * LANES, LANES)],
                                            jnp.float32)
            for e0 in range(LANES):
                e = g * LANES + e0
                bc = lax.gather(
                    vvec, _splat_i(e0)[:, None],
                    lax.GatherDimensionNumbers(
                        offset_dims=(), collapsed_slice_dims=(0,),
                        start_index_map=(0,)),
                    slice_sizes=(1,),
                    mode=lax.GatherScatterMode.PROMISE_IN_BOUNDS)
                rr[e] = rr[e] * bc

    for l in range(NLAYERS):
        src = srcs[l]
        dst = dsts[l]
        if l == 0:
            for k in range(RPT // ZR):
                pltpu.sync_copy(zeros, acc.at[pl.ds(s * RPT + k * ZR, ZR)])
            plsc.subcore_barrier()

        for b in range(NB):
            issue_lin(b, b)

        @pl.loop(0, NCHT, step=NB)
        def _edge_group(g0):
            for b in range(NB):
                wait_lin(b)
                pltpu.async_copy(src.at[gbuf[b].at[0]], rows[b], semG[b])
            for b in range(NB):
                pltpu.make_async_copy(src.at[gbuf[b].at[0]], rows[b],
                                      semG[b]).wait()
                scale_slot(b)
                pltpu.async_copy(rows[b], acc.at[sbuf[b]], semS[b], add=True)
            for b in range(NB):
                pltpu.make_async_copy(rows[b], acc.at[sbuf[b]], semS[b]).wait()

                @pl.when(g0 + NB + b < NCHT)
                def _():
                    issue_lin(b, g0 + NB + b)

        plsc.subcore_barrier()
        for k in range(RPT // ZR):
            r0 = s * RPT + k * ZR
            pltpu.sync_copy(acc.at[pl.ds(r0, ZR)], dst.at[c, pl.ds(r0, ZR)])
        if l < NLAYERS - 1:
            for k in range(RPT // ZR):
                pltpu.sync_copy(zeros, acc.at[pl.ds(s * RPT + k * ZR, ZR)])
        plsc.subcore_barrier()

    bsrcs = (emb.at[c], l1.at[c], l2.at[c], l3.at[c])
    stages = rows[0:4]
    idxb = gbuf[0].at[0]
    outs = (usum, psum, nsum)
    for cc in range(BPT // CH):
        b0 = s * BPT + cc * CH
        for t in range(3):
            pltpu.sync_copy(bidx.at[pl.ds(t * B + b0, CH)], idxb)
            cps = [pltpu.async_copy(bsrcs[l].at[idxb], stages[l], semg)
                   for l in range(4)]
            for cp in cps:
                cp.wait()

            @pl.loop(0, CH, unroll=8)
            def _sum_rows(e):
                stages[0][e] = ((stages[0][e] + stages[1][e]) +
                                (stages[2][e] + stages[3][e]))

            pltpu.sync_copy(stages[0], outs[t].at[c, pl.ds(b0, CH)])


_SLOT_SCRATCH = (
    [pltpu.VMEM((2, CH), jnp.int32) for _ in range(NB)]
    + [pltpu.VMEM((CH,), jnp.int32) for _ in range(NB)]
    + [pltpu.VMEM((CH, HALF), jnp.float32) for _ in range(NB)]
    + [pltpu.SemaphoreType.DMA for _ in range(3 * NB)]
)


@functools.partial(
    pl.kernel,
    out_type=[
        jax.ShapeDtypeStruct((NC, B, HALF), jnp.float32),
        jax.ShapeDtypeStruct((NC, B, HALF), jnp.float32),
        jax.ShapeDtypeStruct((NC, B, HALF), jnp.float32),
        jax.ShapeDtypeStruct((NC, NP, HALF), jnp.float32),
        jax.ShapeDtypeStruct((NC, NP, HALF), jnp.float32),
        jax.ShapeDtypeStruct((NC, NP, HALF), jnp.float32),
    ],
    mesh=plsc.VectorSubcoreMesh(core_axis_name="c", subcore_axis_name="s"),
    compiler_params=pltpu.CompilerParams(use_tc_tiling_on_sc=False, needs_layout_passes=False),
    scratch_types=[pltpu.VMEM_SHARED((NP, HALF), jnp.float32)]
    + _SLOT_SCRATCH
    + [pltpu.SemaphoreType.DMA],
)
def _sc_propagate(emb, egdata, esdata, bidx, zeros, usum, psum, nsum,
                  l1, l2, l3, acc, *scratch):
    _sc_body(emb, egdata, esdata, bidx, zeros, usum, psum, nsum, l1, l2, l3,
             acc, *scratch)


def _loss_body(u_ref, p_ref, n_ref, o_ref):
    u = u_ref[...]
    p = p_ref[...]
    n = n_ref[...]
    t = u * (p - n)
    rowx = jnp.sum(t, axis=1)
    x = (rowx[:B] + rowx[B:]) * (1.0 / 16.0)
    z = jnp.minimum(x, 0.0) - jnp.log(1.0 + jnp.exp(-jnp.abs(x)))
    o_ref[...] = jnp.reshape(-jnp.sum(z) * (1.0 / B), (1, 1))


_loss_call = pl.pallas_call(
    _loss_body,
    out_shape=jax.ShapeDtypeStruct((1, 1), jnp.float32),
)


def kernel(user_table, item_table, adj_val, adj_row, adj_col,
           user_idx, pos_item_idx, neg_item_idx):
    all_emb = jnp.concatenate([user_table, item_table], axis=0)
    halves = jnp.stack([all_emb[:, :HALF], all_emb[:, HALF:]])
    pad = EPAD - E
    spread = jnp.arange(pad, dtype=jnp.int32) % N
    colp = jnp.concatenate([adj_col, spread])
    rowp = jnp.concatenate([adj_row, spread])
    valp = jnp.concatenate([adj_val, jnp.zeros((pad,), jnp.float32)])
    egdata = jnp.stack([jnp.reshape(colp, (-1, CH)),
                        jnp.reshape(valp, (-1, CH)).view(jnp.int32)], axis=1)
    esdata = jnp.reshape(rowp, (-1, CH))
    bidx = jnp.concatenate([user_idx, pos_item_idx + NU, neg_item_idx + NU])
    zeros = jnp.zeros((ZR, HALF), jnp.float32)
    usum, psum, nsum, _, _, _ = _sc_propagate(halves, egdata, esdata, bidx,
                                              zeros)
    loss = _loss_call(jnp.reshape(usum, (NC * B, HALF)),
                      jnp.reshape(psum, (NC * B, HALF)),
                      jnp.reshape(nsum, (NC * B, HALF)))
    return loss[0, 0]
